# baseline (speedup 1.0000x reference)
import functools

import jax
import jax.numpy as jnp
from jax import lax
from jax.experimental import pallas as pl
from jax.experimental.pallas import tpu as pltpu
from jax.experimental.pallas import tpu_sc as plsc

V, H, L = 1000000, 32, 2
B, S = 4096, 200

NC, NS = 2, 16
NW = NC * NS
SPW = B // NW
G = S // 2
HALF = H // 2

CW = 512
NFULL = 999936 // CW
TAIL_C = NFULL * CW
TAIL_N = V - TAIL_C


def _sc_detile(tableT):
    mesh = plsc.VectorSubcoreMesh(core_axis_name="c", subcore_axis_name="s")

    @functools.partial(
        pl.kernel,
        mesh=mesh,
        out_type=jax.ShapeDtypeStruct((V // 4, 128), jnp.float32),
        compiler_params=pltpu.CompilerParams(needs_layout_passes=False),
        scratch_types=[
            pltpu.VMEM((H, CW), jnp.float32),
            pltpu.VMEM((H, CW), jnp.float32),
            pltpu.VMEM((CW // 4, 128), jnp.float32),
            pltpu.VMEM((CW // 4, 128), jnp.float32),
            pltpu.VMEM((H, TAIL_N), jnp.float32),
            pltpu.SemaphoreType.DMA,
            pltpu.SemaphoreType.DMA,
        ],
    )
    def body(t_hbm, out_hbm, in0, in1, ob0, ob1, tail_in, sem_in, sem_out):
        wid = lax.axis_index("s") * NC + lax.axis_index("c")

        iota = lax.iota(jnp.int32, 16)
        row_lo = iota
        row_hi = iota + 16

        def fire_in(c, buf):
            pltpu.async_copy(t_hbm.at[:, pl.ds(c * CW, CW)], buf, sem_in)

        def wait_in(c, buf):
            pltpu.make_async_copy(t_hbm.at[:, pl.ds(c * CW, CW)], buf,
                                  sem_in).wait()

        def regroup(ib, ob):
            def arow(a, _):
                for kg in range(8):
                    rows = row_lo if (16 * kg) % 32 == 0 else row_hi
                    cols = jnp.full((16,), 4 * a + kg // 2, jnp.int32)
                    v = plsc.load_gather(ib, [rows, cols])
                    ob[a, pl.ds(16 * kg, 16)] = v
                return 0

            lax.fori_loop(0, CW // 4, arow, 0)

        def fire_out(c, ob):
            pltpu.async_copy(ob, out_hbm.at[pl.ds(c * (CW // 4), CW // 4)],
                             sem_out)

        def wait_out(c, ob):
            pltpu.make_async_copy(ob, out_hbm.at[pl.ds(c * (CW // 4), CW // 4)],
                                  sem_out).wait()

        NT = (NFULL + NW - 1) // NW

        fire_in(wid, in0)

        def step(t, c, ib, ob, other_ib):
            nxt = c + NW

            @pl.when(nxt < NFULL)
            def _():
                fire_in(nxt, other_ib)

            wait_in(c, ib)
            @pl.when(t >= 2)
            def _():
                wait_out(c, ob)

            regroup(ib, ob)
            fire_out(c, ob)

        def loop_body(u, _):
            t0 = 2 * u
            c0 = t0 * NW + wid

            @pl.when(c0 < NFULL)
            def _():
                step(t0, c0, in0, ob0, in1)

            @pl.when(c0 + NW < NFULL)
            def _():
                step(t0 + 1, c0 + NW, in1, ob1, in0)

            return 0

        lax.fori_loop(0, (NT + 1) // 2, loop_body, 0)

        wait_out(wid, ob0)
        wait_out(wid, ob1)

        @pl.when(wid == 4)
        def _():
            pltpu.sync_copy(t_hbm.at[:, pl.ds(TAIL_C, TAIL_N)], tail_in)

            def arow(a, _):
                for kg in range(8):
                    rows = row_lo if (16 * kg) % 32 == 0 else row_hi
                    cols = jnp.full((16,), 4 * a + kg // 2, jnp.int32)
                    v = plsc.load_gather(tail_in, [rows, cols])
                    ob0[a, pl.ds(16 * kg, 16)] = v
                return 0

            lax.fori_loop(0, TAIL_N // 4, arow, 0)
            pltpu.sync_copy(ob0.at[pl.ds(0, TAIL_N // 4)],
                            out_hbm.at[pl.ds(TAIL_C // 4, TAIL_N // 4)])

    return body(tableT)


def _sc_emb_sum(ids2d, table):
    mesh = plsc.VectorSubcoreMesh(core_axis_name="c", subcore_axis_name="s")

    @functools.partial(
        pl.kernel,
        mesh=mesh,
        out_type=jax.ShapeDtypeStruct((B, H), jnp.float32),
        compiler_params=pltpu.CompilerParams(use_tc_tiling_on_sc=False),
        scratch_types=[
            pltpu.VMEM((2 * SPW, G), jnp.int32),
            pltpu.VMEM((G, H), jnp.float32),
            pltpu.VMEM((G, H), jnp.float32),
            pltpu.VMEM((G, H), jnp.float32),
            pltpu.VMEM((G, H), jnp.float32),
            pltpu.VMEM((SPW, H), jnp.float32),
            pltpu.SemaphoreType.DMA,
            pltpu.SemaphoreType.DMA,
        ],
    )
    def body(ids_hbm, table_hbm, out_hbm, ids_v, r0a, r0b, r1a, r1b, sums_v,
             sem0, sem1):
        wid = lax.axis_index("s") * NC + lax.axis_index("c")
        base = wid * (2 * SPW)
        pltpu.sync_copy(ids_hbm.at[pl.ds(base, 2 * SPW)], ids_v)

        def fire(s, ra, rb, sem):
            pltpu.async_copy(table_hbm.at[ids_v.at[2 * s]], ra, sem)
            pltpu.async_copy(table_hbm.at[ids_v.at[2 * s + 1]], rb, sem)

        def drain(s, ra, rb, sem):
            pltpu.make_async_copy(table_hbm.at[ids_v.at[2 * s]], ra, sem).wait()
            pltpu.make_async_copy(table_hbm.at[ids_v.at[2 * s + 1]], rb,
                                  sem).wait()

        def accum(buf, carry):
            def inner(i, c):
                a0, a1, a2, a3 = c
                for j in range(0, 20, 2):
                    r = i * 20 + j
                    a0 = a0 + buf[r, pl.ds(0, HALF)]
                    a1 = a1 + buf[r, pl.ds(HALF, HALF)]
                    a2 = a2 + buf[r + 1, pl.ds(0, HALF)]
                    a3 = a3 + buf[r + 1, pl.ds(HALF, HALF)]
                return (a0, a1, a2, a3)

            return lax.fori_loop(0, G // 20, inner, carry)

        def do_sample(s, ra, rb, sem):
            drain(s, ra, rb, sem)
            z = jnp.zeros((HALF,), jnp.float32)
            c = accum(ra, (z, z, z, z))
            c = accum(rb, c)
            sums_v[s, pl.ds(0, HALF)] = c[0] + c[2]
            sums_v[s, pl.ds(HALF, HALF)] = c[1] + c[3]

        fire(0, r0a, r0b, sem0)

        def loop_body(t, _):
            s0 = 2 * t
            fire(s0 + 1, r1a, r1b, sem1)
            do_sample(s0, r0a, r0b, sem0)

            @pl.when(s0 + 2 < SPW)
            def _():
                fire(s0 + 2, r0a, r0b, sem0)

            do_sample(s0 + 1, r1a, r1b, sem1)
            return 0

        lax.fori_loop(0, SPW // 2, loop_body, 0)
        pltpu.sync_copy(sums_v, out_hbm.at[pl.ds(wid * SPW, SPW)])

    return body(ids2d, table)


def _tc_head(sums, tt, type_table, Wp, bp):
    def body(sums_ref, tt_ref, type_ref, w_ref, b_ref, out_ref):
        c1 = jnp.sum(tt_ref[...].astype(jnp.float32), axis=1, keepdims=True)
        t0 = type_ref[0:1, :]
        t1 = type_ref[1:2, :]
        inv_s = jnp.float32(1.0 / S)
        pooled = (sums_ref[...] + (jnp.float32(S) - c1) * t0 + c1 * t1) * inv_s
        out_ref[...] = (
            jnp.dot(pooled, w_ref[...], preferred_element_type=jnp.float32)
            + b_ref[...]
        )

    return pl.pallas_call(
        body,
        out_shape=jax.ShapeDtypeStruct((B, 128), jnp.float32),
    )(sums, tt, type_table, Wp, bp)


def kernel(input_ids, attention_mask, token_type_ids, emb_table, type_table,
           W, b):
    del attention_mask
    t4 = _sc_detile(emb_table.T)
    table_lin = t4.reshape(V, H)
    ids2d = input_ids.astype(jnp.int32).reshape(NW * 2 * SPW, G)
    sums = _sc_emb_sum(ids2d, table_lin)
    Wp = jnp.pad(W.astype(jnp.float32), ((0, 0), (0, 128 - L)))
    bp = jnp.pad(b.astype(jnp.float32), (0, 128 - L)).reshape(1, 128)
    logits = _tc_head(sums, token_type_ids.astype(jnp.int32), type_table, Wp,
                      bp)
    return logits[:, :L]


# speedup vs baseline: 1.1972x; 1.1972x over previous
import functools

import jax
import jax.numpy as jnp
from jax import lax
from jax.experimental import pallas as pl
from jax.experimental.pallas import tpu as pltpu
from jax.experimental.pallas import tpu_sc as plsc

V, H, L = 1000000, 32, 2
B, S = 4096, 200

NC, NS = 2, 16
NW = NC * NS
SPW = B // NW
G = S // 2
HALF = H // 2

CW = 512
NFULL = 999936 // CW
TAIL_C = NFULL * CW
TAIL_N = V - TAIL_C


def _sc_detile(tableT):
    mesh = plsc.VectorSubcoreMesh(core_axis_name="c", subcore_axis_name="s")

    @functools.partial(
        pl.kernel,
        mesh=mesh,
        out_type=jax.ShapeDtypeStruct((V // 4, 128), jnp.float32),
        compiler_params=pltpu.CompilerParams(needs_layout_passes=False),
        scratch_types=[
            pltpu.VMEM((H, CW), jnp.float32),
            pltpu.VMEM((H, CW), jnp.float32),
            pltpu.VMEM((CW // 4, 128), jnp.float32),
            pltpu.VMEM((CW // 4, 128), jnp.float32),
            pltpu.VMEM((H, TAIL_N), jnp.float32),
            pltpu.SemaphoreType.DMA,
            pltpu.SemaphoreType.DMA,
        ],
    )
    def body(t_hbm, out_hbm, in0, in1, ob0, ob1, tail_in, sem_in, sem_out):
        wid = lax.axis_index("s") * NC + lax.axis_index("c")

        iota = lax.iota(jnp.int32, 16)
        rowbase = iota // 4
        colbase = (iota % 4) * 32

        def fire_in(c, buf):
            pltpu.async_copy(t_hbm.at[:, pl.ds(c * CW, CW)], buf, sem_in)

        def wait_in(c, buf):
            pltpu.make_async_copy(t_hbm.at[:, pl.ds(c * CW, CW)], buf,
                                  sem_in).wait()

        def regroup(ib, ob):
            def hbody(h, _):
                colidx = colbase + h
                for m in range(CW // 16):
                    v = ib[h, pl.ds(16 * m, 16)]
                    plsc.store_scatter(ob, [rowbase + 4 * m, colidx], v)
                return 0

            lax.fori_loop(0, H, hbody, 0)

        def fire_out(c, ob):
            pltpu.async_copy(ob, out_hbm.at[pl.ds(c * (CW // 4), CW // 4)],
                             sem_out)

        def wait_out(c, ob):
            pltpu.make_async_copy(ob, out_hbm.at[pl.ds(c * (CW // 4), CW // 4)],
                                  sem_out).wait()

        NT = (NFULL + NW - 1) // NW

        fire_in(wid, in0)

        def step(t, c, ib, ob, other_ib):
            nxt = c + NW

            @pl.when(nxt < NFULL)
            def _():
                fire_in(nxt, other_ib)

            wait_in(c, ib)
            @pl.when(t >= 2)
            def _():
                wait_out(c, ob)

            regroup(ib, ob)
            fire_out(c, ob)

        def loop_body(u, _):
            t0 = 2 * u
            c0 = t0 * NW + wid

            @pl.when(c0 < NFULL)
            def _():
                step(t0, c0, in0, ob0, in1)

            @pl.when(c0 + NW < NFULL)
            def _():
                step(t0 + 1, c0 + NW, in1, ob1, in0)

            return 0

        lax.fori_loop(0, (NT + 1) // 2, loop_body, 0)

        wait_out(wid, ob0)
        wait_out(wid, ob1)

        @pl.when(wid == 4)
        def _():
            pltpu.sync_copy(t_hbm.at[:, pl.ds(TAIL_C, TAIL_N)], tail_in)

            def hbody(h, _):
                colidx = colbase + h
                for m in range(TAIL_N // 16):
                    v = tail_in[h, pl.ds(16 * m, 16)]
                    plsc.store_scatter(ob0, [rowbase + 4 * m, colidx], v)
                return 0

            lax.fori_loop(0, H, hbody, 0)
            pltpu.sync_copy(ob0.at[pl.ds(0, TAIL_N // 4)],
                            out_hbm.at[pl.ds(TAIL_C // 4, TAIL_N // 4)])

    return body(tableT)


def _sc_emb_sum(ids2d, table):
    mesh = plsc.VectorSubcoreMesh(core_axis_name="c", subcore_axis_name="s")

    @functools.partial(
        pl.kernel,
        mesh=mesh,
        out_type=jax.ShapeDtypeStruct((B, H), jnp.float32),
        compiler_params=pltpu.CompilerParams(use_tc_tiling_on_sc=False),
        scratch_types=[
            pltpu.VMEM((2 * SPW, G), jnp.int32),
            pltpu.VMEM((G, H), jnp.float32),
            pltpu.VMEM((G, H), jnp.float32),
            pltpu.VMEM((G, H), jnp.float32),
            pltpu.VMEM((G, H), jnp.float32),
            pltpu.VMEM((SPW, H), jnp.float32),
            pltpu.SemaphoreType.DMA,
            pltpu.SemaphoreType.DMA,
        ],
    )
    def body(ids_hbm, table_hbm, out_hbm, ids_v, r0a, r0b, r1a, r1b, sums_v,
             sem0, sem1):
        wid = lax.axis_index("s") * NC + lax.axis_index("c")
        base = wid * (2 * SPW)
        pltpu.sync_copy(ids_hbm.at[pl.ds(base, 2 * SPW)], ids_v)

        def fire(s, ra, rb, sem):
            pltpu.async_copy(table_hbm.at[ids_v.at[2 * s]], ra, sem)
            pltpu.async_copy(table_hbm.at[ids_v.at[2 * s + 1]], rb, sem)

        def drain(s, ra, rb, sem):
            pltpu.make_async_copy(table_hbm.at[ids_v.at[2 * s]], ra, sem).wait()
            pltpu.make_async_copy(table_hbm.at[ids_v.at[2 * s + 1]], rb,
                                  sem).wait()

        def accum(buf, carry):
            def inner(i, c):
                a0, a1, a2, a3 = c
                for j in range(0, 20, 2):
                    r = i * 20 + j
                    a0 = a0 + buf[r, pl.ds(0, HALF)]
                    a1 = a1 + buf[r, pl.ds(HALF, HALF)]
                    a2 = a2 + buf[r + 1, pl.ds(0, HALF)]
                    a3 = a3 + buf[r + 1, pl.ds(HALF, HALF)]
                return (a0, a1, a2, a3)

            return lax.fori_loop(0, G // 20, inner, carry)

        def do_sample(s, ra, rb, sem):
            drain(s, ra, rb, sem)
            z = jnp.zeros((HALF,), jnp.float32)
            c = accum(ra, (z, z, z, z))
            c = accum(rb, c)
            sums_v[s, pl.ds(0, HALF)] = c[0] + c[2]
            sums_v[s, pl.ds(HALF, HALF)] = c[1] + c[3]

        fire(0, r0a, r0b, sem0)

        def loop_body(t, _):
            s0 = 2 * t
            fire(s0 + 1, r1a, r1b, sem1)
            do_sample(s0, r0a, r0b, sem0)

            @pl.when(s0 + 2 < SPW)
            def _():
                fire(s0 + 2, r0a, r0b, sem0)

            do_sample(s0 + 1, r1a, r1b, sem1)
            return 0

        lax.fori_loop(0, SPW // 2, loop_body, 0)
        pltpu.sync_copy(sums_v, out_hbm.at[pl.ds(wid * SPW, SPW)])

    return body(ids2d, table)


def _tc_head(sums, tt, type_table, Wp, bp):
    def body(sums_ref, tt_ref, type_ref, w_ref, b_ref, out_ref):
        c1 = jnp.sum(tt_ref[...].astype(jnp.float32), axis=1, keepdims=True)
        t0 = type_ref[0:1, :]
        t1 = type_ref[1:2, :]
        inv_s = jnp.float32(1.0 / S)
        pooled = (sums_ref[...] + (jnp.float32(S) - c1) * t0 + c1 * t1) * inv_s
        out_ref[...] = (
            jnp.dot(pooled, w_ref[...], preferred_element_type=jnp.float32)
            + b_ref[...]
        )

    return pl.pallas_call(
        body,
        out_shape=jax.ShapeDtypeStruct((B, 128), jnp.float32),
    )(sums, tt, type_table, Wp, bp)


def kernel(input_ids, attention_mask, token_type_ids, emb_table, type_table,
           W, b):
    del attention_mask
    t4 = _sc_detile(emb_table.T)
    table_lin = t4.reshape(V, H)
    ids2d = input_ids.astype(jnp.int32).reshape(NW * 2 * SPW, G)
    sums = _sc_emb_sum(ids2d, table_lin)
    Wp = jnp.pad(W.astype(jnp.float32), ((0, 0), (0, 128 - L)))
    bp = jnp.pad(b.astype(jnp.float32), (0, 128 - L)).reshape(1, 128)
    logits = _tc_head(sums, token_type_ids.astype(jnp.int32), type_table, Wp,
                      bp)
    return logits[:, :L]


# speedup vs baseline: 1.2028x; 1.0047x over previous
import functools

import jax
import jax.numpy as jnp
from jax import lax
from jax.experimental import pallas as pl
from jax.experimental.pallas import tpu as pltpu
from jax.experimental.pallas import tpu_sc as plsc

V, H, L = 1000000, 32, 2
B, S = 4096, 200

NC, NS = 2, 16
NW = NC * NS
SPW = B // NW
G = S // 2
HALF = H // 2

CW = 512
NFULL = 999936 // CW
TAIL_C = NFULL * CW
TAIL_N = V - TAIL_C


def _sc_detile(tableT):
    mesh = plsc.VectorSubcoreMesh(core_axis_name="c", subcore_axis_name="s")

    @functools.partial(
        pl.kernel,
        mesh=mesh,
        out_type=jax.ShapeDtypeStruct((V * H,), jnp.float32),
        compiler_params=pltpu.CompilerParams(needs_layout_passes=False),
        scratch_types=[
            pltpu.VMEM((H, CW), jnp.float32),
            pltpu.VMEM((H, CW), jnp.float32),
            pltpu.VMEM((CW * H,), jnp.float32),
            pltpu.VMEM((CW * H,), jnp.float32),
            pltpu.VMEM((H, TAIL_N), jnp.float32),
            pltpu.SemaphoreType.DMA,
            pltpu.SemaphoreType.DMA,
        ],
    )
    def body(t_hbm, out_hbm, in0, in1, ob0, ob1, tail_in, sem_in, sem_out):
        wid = lax.axis_index("s") * NC + lax.axis_index("c")

        iota = lax.iota(jnp.int32, 16)
        flatbase = (iota // 4) * 128 + (iota % 4) * 32

        def fire_in(c, buf):
            pltpu.async_copy(t_hbm.at[:, pl.ds(c * CW, CW)], buf, sem_in)

        def wait_in(c, buf):
            pltpu.make_async_copy(t_hbm.at[:, pl.ds(c * CW, CW)], buf,
                                  sem_in).wait()

        def regroup_h(ib, ob, h, nm):
            for m0 in range(0, nm, 8):
                ms = list(range(m0, min(m0 + 8, nm)))
                vs = [ib[h, pl.ds(16 * m, 16)] for m in ms]
                idxs = [flatbase + (512 * m + h) for m in ms]
                for j in range(len(ms)):
                    plsc.store_scatter(ob, [idxs[j]], vs[j])

        def regroup(ib, ob):
            def hbody(h, _):
                regroup_h(ib, ob, h, CW // 16)
                return 0

            lax.fori_loop(0, H, hbody, 0)

        def fire_out(c, ob):
            pltpu.async_copy(ob, out_hbm.at[pl.ds(c * (CW * H), CW * H)],
                             sem_out)

        def wait_out(c, ob):
            pltpu.make_async_copy(ob, out_hbm.at[pl.ds(c * (CW * H), CW * H)],
                                  sem_out).wait()

        NT = (NFULL + NW - 1) // NW

        fire_in(wid, in0)

        def step(t, c, ib, ob, other_ib):
            nxt = c + NW

            @pl.when(nxt < NFULL)
            def _():
                fire_in(nxt, other_ib)

            wait_in(c, ib)
            @pl.when(t >= 2)
            def _():
                wait_out(c, ob)

            regroup(ib, ob)
            fire_out(c, ob)

        def loop_body(u, _):
            t0 = 2 * u
            c0 = t0 * NW + wid

            @pl.when(c0 < NFULL)
            def _():
                step(t0, c0, in0, ob0, in1)

            @pl.when(c0 + NW < NFULL)
            def _():
                step(t0 + 1, c0 + NW, in1, ob1, in0)

            return 0

        lax.fori_loop(0, (NT + 1) // 2, loop_body, 0)

        wait_out(wid, ob0)
        wait_out(wid, ob1)

        @pl.when(wid == 4)
        def _():
            pltpu.sync_copy(t_hbm.at[:, pl.ds(TAIL_C, TAIL_N)], tail_in)

            def hbody(h, _):
                regroup_h(tail_in, ob0, h, TAIL_N // 16)
                return 0

            lax.fori_loop(0, H, hbody, 0)
            pltpu.sync_copy(ob0.at[pl.ds(0, TAIL_N * H)],
                            out_hbm.at[pl.ds(TAIL_C * H, TAIL_N * H)])

    return body(tableT)


def _sc_emb_sum(ids2d, table):
    mesh = plsc.VectorSubcoreMesh(core_axis_name="c", subcore_axis_name="s")

    @functools.partial(
        pl.kernel,
        mesh=mesh,
        out_type=jax.ShapeDtypeStruct((B, H), jnp.float32),
        compiler_params=pltpu.CompilerParams(use_tc_tiling_on_sc=False),
        scratch_types=[
            pltpu.VMEM((2 * SPW, G), jnp.int32),
            pltpu.VMEM((G, H), jnp.float32),
            pltpu.VMEM((G, H), jnp.float32),
            pltpu.VMEM((G, H), jnp.float32),
            pltpu.VMEM((G, H), jnp.float32),
            pltpu.VMEM((SPW, H), jnp.float32),
            pltpu.SemaphoreType.DMA,
            pltpu.SemaphoreType.DMA,
        ],
    )
    def body(ids_hbm, table_hbm, out_hbm, ids_v, r0a, r0b, r1a, r1b, sums_v,
             sem0, sem1):
        wid = lax.axis_index("s") * NC + lax.axis_index("c")
        base = wid * (2 * SPW)
        pltpu.sync_copy(ids_hbm.at[pl.ds(base, 2 * SPW)], ids_v)

        def fire(s, ra, rb, sem):
            pltpu.async_copy(table_hbm.at[ids_v.at[2 * s]], ra, sem)
            pltpu.async_copy(table_hbm.at[ids_v.at[2 * s + 1]], rb, sem)

        def drain(s, ra, rb, sem):
            pltpu.make_async_copy(table_hbm.at[ids_v.at[2 * s]], ra, sem).wait()
            pltpu.make_async_copy(table_hbm.at[ids_v.at[2 * s + 1]], rb,
                                  sem).wait()

        def accum(buf, carry):
            def inner(i, c):
                a0, a1, a2, a3 = c
                for j in range(0, 20, 2):
                    r = i * 20 + j
                    a0 = a0 + buf[r, pl.ds(0, HALF)]
                    a1 = a1 + buf[r, pl.ds(HALF, HALF)]
                    a2 = a2 + buf[r + 1, pl.ds(0, HALF)]
                    a3 = a3 + buf[r + 1, pl.ds(HALF, HALF)]
                return (a0, a1, a2, a3)

            return lax.fori_loop(0, G // 20, inner, carry)

        def do_sample(s, ra, rb, sem):
            drain(s, ra, rb, sem)
            z = jnp.zeros((HALF,), jnp.float32)
            c = accum(ra, (z, z, z, z))
            c = accum(rb, c)
            sums_v[s, pl.ds(0, HALF)] = c[0] + c[2]
            sums_v[s, pl.ds(HALF, HALF)] = c[1] + c[3]

        fire(0, r0a, r0b, sem0)

        def loop_body(t, _):
            s0 = 2 * t
            fire(s0 + 1, r1a, r1b, sem1)
            do_sample(s0, r0a, r0b, sem0)

            @pl.when(s0 + 2 < SPW)
            def _():
                fire(s0 + 2, r0a, r0b, sem0)

            do_sample(s0 + 1, r1a, r1b, sem1)
            return 0

        lax.fori_loop(0, SPW // 2, loop_body, 0)
        pltpu.sync_copy(sums_v, out_hbm.at[pl.ds(wid * SPW, SPW)])

    return body(ids2d, table)


def _tc_head(sums, tt, type_table, Wp, bp):
    def body(sums_ref, tt_ref, type_ref, w_ref, b_ref, out_ref):
        c1 = jnp.sum(tt_ref[...].astype(jnp.float32), axis=1, keepdims=True)
        t0 = type_ref[0:1, :]
        t1 = type_ref[1:2, :]
        inv_s = jnp.float32(1.0 / S)
        pooled = (sums_ref[...] + (jnp.float32(S) - c1) * t0 + c1 * t1) * inv_s
        out_ref[...] = (
            jnp.dot(pooled, w_ref[...], preferred_element_type=jnp.float32)
            + b_ref[...]
        )

    return pl.pallas_call(
        body,
        out_shape=jax.ShapeDtypeStruct((B, 128), jnp.float32),
    )(sums, tt, type_table, Wp, bp)


def kernel(input_ids, attention_mask, token_type_ids, emb_table, type_table,
           W, b):
    del attention_mask
    t4 = _sc_detile(emb_table.T)
    table_lin = t4.reshape(V, H)
    ids2d = input_ids.astype(jnp.int32).reshape(NW * 2 * SPW, G)
    sums = _sc_emb_sum(ids2d, table_lin)
    Wp = jnp.pad(W.astype(jnp.float32), ((0, 0), (0, 128 - L)))
    bp = jnp.pad(b.astype(jnp.float32), (0, 128 - L)).reshape(1, 128)
    logits = _tc_head(sums, token_type_ids.astype(jnp.int32), type_table, Wp,
                      bp)
    return logits[:, :L]


# speedup vs baseline: 1.5514x; 1.2898x over previous
import functools

import jax
import jax.numpy as jnp
from jax import lax
from jax.experimental import pallas as pl
from jax.experimental.pallas import tpu as pltpu
from jax.experimental.pallas import tpu_sc as plsc

V, H, L = 1000000, 32, 2
B, S = 4096, 200

NC, NS = 2, 16
NW = NC * NS
SPW = B // NW
G = S // 2
HALF = H // 2

CW = 512
NFULL = 999936 // CW
TAIL_C = NFULL * CW
TAIL_N = V - TAIL_C


def _sc_detile(tableT):
    mesh = plsc.VectorSubcoreMesh(core_axis_name="c", subcore_axis_name="s")

    @functools.partial(
        pl.kernel,
        mesh=mesh,
        out_type=jax.ShapeDtypeStruct((V * H,), jnp.float32),
        compiler_params=pltpu.CompilerParams(needs_layout_passes=False),
        scratch_types=[
            pltpu.VMEM((H, CW + 1), jnp.float32),
            pltpu.VMEM((H, CW + 1), jnp.float32),
            pltpu.VMEM((CW * H,), jnp.float32),
            pltpu.VMEM((CW * H,), jnp.float32),
            pltpu.VMEM((H, TAIL_N), jnp.float32),
            pltpu.SemaphoreType.DMA,
            pltpu.SemaphoreType.DMA,
        ],
    )
    def body(t_hbm, out_hbm, in0, in1, ob0, ob1, tail_in, sem_in, sem_out):
        wid = lax.axis_index("s") * NC + lax.axis_index("c")

        iota = lax.iota(jnp.int32, 16)
        row_lo = iota
        row_hi = iota + 16

        def fire_in(c, buf):
            pltpu.async_copy(t_hbm.at[:, pl.ds(c * CW, CW)],
                             buf.at[:, pl.ds(0, CW)], sem_in)

        def wait_in(c, buf):
            pltpu.make_async_copy(t_hbm.at[:, pl.ds(c * CW, CW)],
                                  buf.at[:, pl.ds(0, CW)], sem_in).wait()

        def regroup_js(ib, ob, j0, njs):
            js = list(range(njs))
            cols = [jnp.full((16,), j0 + jj, jnp.int32) for jj in js]
            vlo = [plsc.load_gather(ib, [row_lo, cols[jj]]) for jj in js]
            vhi = [plsc.load_gather(ib, [row_hi, cols[jj]]) for jj in js]
            for jj in js:
                o = (j0 // 4) * 128 + jj * 32
                ob[pl.ds(o, 16)] = vlo[jj]
                ob[pl.ds(o + 16, 16)] = vhi[jj]

        def regroup(ib, ob):
            def jbody(t, _):
                regroup_js(ib, ob, 4 * t, 4)
                return 0

            lax.fori_loop(0, CW // 4, jbody, 0)

        def fire_out(c, ob):
            pltpu.async_copy(ob, out_hbm.at[pl.ds(c * (CW * H), CW * H)],
                             sem_out)

        def wait_out(c, ob):
            pltpu.make_async_copy(ob, out_hbm.at[pl.ds(c * (CW * H), CW * H)],
                                  sem_out).wait()

        NT = (NFULL + NW - 1) // NW

        fire_in(wid, in0)

        def step(t, c, ib, ob, other_ib):
            nxt = c + NW

            @pl.when(nxt < NFULL)
            def _():
                fire_in(nxt, other_ib)

            wait_in(c, ib)
            @pl.when(t >= 2)
            def _():
                wait_out(c, ob)

            regroup(ib, ob)
            fire_out(c, ob)

        def loop_body(u, _):
            t0 = 2 * u
            c0 = t0 * NW + wid

            @pl.when(c0 < NFULL)
            def _():
                step(t0, c0, in0, ob0, in1)

            @pl.when(c0 + NW < NFULL)
            def _():
                step(t0 + 1, c0 + NW, in1, ob1, in0)

            return 0

        lax.fori_loop(0, (NT + 1) // 2, loop_body, 0)

        wait_out(wid, ob0)
        wait_out(wid, ob1)

        @pl.when(wid == 4)
        def _():
            pltpu.sync_copy(t_hbm.at[:, pl.ds(TAIL_C, TAIL_N)], tail_in)

            def jbody(t, _):
                regroup_js(tail_in, ob0, 4 * t, 4)
                return 0

            lax.fori_loop(0, TAIL_N // 4, jbody, 0)
            pltpu.sync_copy(ob0.at[pl.ds(0, TAIL_N * H)],
                            out_hbm.at[pl.ds(TAIL_C * H, TAIL_N * H)])

    return body(tableT)


def _sc_emb_sum(ids2d, table):
    mesh = plsc.VectorSubcoreMesh(core_axis_name="c", subcore_axis_name="s")

    @functools.partial(
        pl.kernel,
        mesh=mesh,
        out_type=jax.ShapeDtypeStruct((B, H), jnp.float32),
        compiler_params=pltpu.CompilerParams(use_tc_tiling_on_sc=False),
        scratch_types=[
            pltpu.VMEM((2 * SPW, G), jnp.int32),
            pltpu.VMEM((G, H), jnp.float32),
            pltpu.VMEM((G, H), jnp.float32),
            pltpu.VMEM((G, H), jnp.float32),
            pltpu.VMEM((G, H), jnp.float32),
            pltpu.VMEM((SPW, H), jnp.float32),
            pltpu.SemaphoreType.DMA,
            pltpu.SemaphoreType.DMA,
        ],
    )
    def body(ids_hbm, table_hbm, out_hbm, ids_v, r0a, r0b, r1a, r1b, sums_v,
             sem0, sem1):
        wid = lax.axis_index("s") * NC + lax.axis_index("c")
        base = wid * (2 * SPW)
        pltpu.sync_copy(ids_hbm.at[pl.ds(base, 2 * SPW)], ids_v)

        def fire(s, ra, rb, sem):
            pltpu.async_copy(table_hbm.at[ids_v.at[2 * s]], ra, sem)
            pltpu.async_copy(table_hbm.at[ids_v.at[2 * s + 1]], rb, sem)

        def drain(s, ra, rb, sem):
            pltpu.make_async_copy(table_hbm.at[ids_v.at[2 * s]], ra, sem).wait()
            pltpu.make_async_copy(table_hbm.at[ids_v.at[2 * s + 1]], rb,
                                  sem).wait()

        def accum(buf, carry):
            def inner(i, c):
                a0, a1, a2, a3 = c
                for j in range(0, 20, 2):
                    r = i * 20 + j
                    a0 = a0 + buf[r, pl.ds(0, HALF)]
                    a1 = a1 + buf[r, pl.ds(HALF, HALF)]
                    a2 = a2 + buf[r + 1, pl.ds(0, HALF)]
                    a3 = a3 + buf[r + 1, pl.ds(HALF, HALF)]
                return (a0, a1, a2, a3)

            return lax.fori_loop(0, G // 20, inner, carry)

        def do_sample(s, ra, rb, sem):
            drain(s, ra, rb, sem)
            z = jnp.zeros((HALF,), jnp.float32)
            c = accum(ra, (z, z, z, z))
            c = accum(rb, c)
            sums_v[s, pl.ds(0, HALF)] = c[0] + c[2]
            sums_v[s, pl.ds(HALF, HALF)] = c[1] + c[3]

        fire(0, r0a, r0b, sem0)

        def loop_body(t, _):
            s0 = 2 * t
            fire(s0 + 1, r1a, r1b, sem1)
            do_sample(s0, r0a, r0b, sem0)

            @pl.when(s0 + 2 < SPW)
            def _():
                fire(s0 + 2, r0a, r0b, sem0)

            do_sample(s0 + 1, r1a, r1b, sem1)
            return 0

        lax.fori_loop(0, SPW // 2, loop_body, 0)
        pltpu.sync_copy(sums_v, out_hbm.at[pl.ds(wid * SPW, SPW)])

    return body(ids2d, table)


def _tc_head(sums, tt, type_table, Wp, bp):
    def body(sums_ref, tt_ref, type_ref, w_ref, b_ref, out_ref):
        c1 = jnp.sum(tt_ref[...].astype(jnp.float32), axis=1, keepdims=True)
        t0 = type_ref[0:1, :]
        t1 = type_ref[1:2, :]
        inv_s = jnp.float32(1.0 / S)
        pooled = (sums_ref[...] + (jnp.float32(S) - c1) * t0 + c1 * t1) * inv_s
        out_ref[...] = (
            jnp.dot(pooled, w_ref[...], preferred_element_type=jnp.float32)
            + b_ref[...]
        )

    return pl.pallas_call(
        body,
        out_shape=jax.ShapeDtypeStruct((B, 128), jnp.float32),
    )(sums, tt, type_table, Wp, bp)


def kernel(input_ids, attention_mask, token_type_ids, emb_table, type_table,
           W, b):
    del attention_mask
    t4 = _sc_detile(emb_table.T)
    table_lin = t4.reshape(V, H)
    ids2d = input_ids.astype(jnp.int32).reshape(NW * 2 * SPW, G)
    sums = _sc_emb_sum(ids2d, table_lin)
    Wp = jnp.pad(W.astype(jnp.float32), ((0, 0), (0, 128 - L)))
    bp = jnp.pad(b.astype(jnp.float32), (0, 128 - L)).reshape(1, 128)
    logits = _tc_head(sums, token_type_ids.astype(jnp.int32), type_table, Wp,
                      bp)
    return logits[:, :L]


# speedup vs baseline: 2.2444x; 1.4467x over previous
import functools

import jax
import jax.numpy as jnp
from jax import lax
from jax.experimental import pallas as pl
from jax.experimental.pallas import tpu as pltpu
from jax.experimental.pallas import tpu_sc as plsc

V, H, L = 1000000, 32, 2
B, S = 4096, 200

NC, NS = 2, 16
NW = NC * NS
SPW = B // NW
G = S // 2
HALF = H // 2

PH = H // 2
CW = 512
NFULL = 999936 // CW
TAIL_C = NFULL * CW
TAIL_N = V - TAIL_C


def _sc_detile(tableT):
    mesh = plsc.VectorSubcoreMesh(core_axis_name="c", subcore_axis_name="s")

    @functools.partial(
        pl.kernel,
        mesh=mesh,
        out_type=jax.ShapeDtypeStruct((V * PH,), jnp.int32),
        compiler_params=pltpu.CompilerParams(needs_layout_passes=False),
        scratch_types=[
            pltpu.VMEM((H, CW + 1), jnp.float32),
            pltpu.VMEM((H, CW + 1), jnp.float32),
            pltpu.VMEM((PH, CW + 1), jnp.int32),
            pltpu.VMEM((CW * PH,), jnp.int32),
            pltpu.VMEM((CW * PH,), jnp.int32),
            pltpu.VMEM((H, TAIL_N), jnp.float32),
            pltpu.SemaphoreType.DMA,
            pltpu.SemaphoreType.DMA,
        ],
    )
    def body(t_hbm, out_hbm, in0, in1, pb, ob0, ob1, tail_in, sem_in, sem_out):
        wid = lax.axis_index("s") * NC + lax.axis_index("c")

        iota = lax.iota(jnp.int32, 16)
        row_all = iota
        half = jnp.int32(0x8000)
        himask = jnp.int32(-65536)

        def fire_in(c, buf):
            pltpu.async_copy(t_hbm.at[:, pl.ds(c * CW, CW)],
                             buf.at[:, pl.ds(0, CW)], sem_in)

        def wait_in(c, buf):
            pltpu.make_async_copy(t_hbm.at[:, pl.ds(c * CW, CW)],
                                  buf.at[:, pl.ds(0, CW)], sem_in).wait()

        def pack_pass(ib, nm):
            def hbody(h, _):
                for m0 in range(0, nm, 4):
                    ms = list(range(m0, min(m0 + 4, nm)))
                    vlo = [ib[h, pl.ds(16 * m, 16)] for m in ms]
                    vhi = [ib[h + PH, pl.ds(16 * m, 16)] for m in ms]
                    for k, m in enumerate(ms):
                        li = lax.bitcast_convert_type(vlo[k], jnp.int32)
                        hi = lax.bitcast_convert_type(vhi[k], jnp.int32)
                        w = lax.shift_right_logical(li + half, 16) | (
                            (hi + half) & himask)
                        pb[h, pl.ds(16 * m, 16)] = w
                return 0

            lax.fori_loop(0, PH, hbody, 0)

        def col_pass(ob, nj):
            def jbody(t, _):
                j0 = 4 * t
                cols = [jnp.full((16,), j0 + jj, jnp.int32) for jj in range(4)]
                vs = [plsc.load_gather(pb, [row_all, cols[jj]])
                      for jj in range(4)]
                for jj in range(4):
                    ob[pl.ds(j0 * PH + jj * 16, 16)] = vs[jj]
                return 0

            lax.fori_loop(0, nj // 4, jbody, 0)

        def regroup(ib, ob):
            pack_pass(ib, CW // 16)
            col_pass(ob, CW)

        def fire_out(c, ob):
            pltpu.async_copy(ob, out_hbm.at[pl.ds(c * (CW * PH), CW * PH)],
                             sem_out)

        def wait_out(c, ob):
            pltpu.make_async_copy(ob,
                                  out_hbm.at[pl.ds(c * (CW * PH), CW * PH)],
                                  sem_out).wait()

        NT = (NFULL + NW - 1) // NW

        fire_in(wid, in0)

        def step(t, c, ib, ob, other_ib):
            nxt = c + NW

            @pl.when(nxt < NFULL)
            def _():
                fire_in(nxt, other_ib)

            wait_in(c, ib)
            @pl.when(t >= 2)
            def _():
                wait_out(c, ob)

            regroup(ib, ob)
            fire_out(c, ob)

        def loop_body(u, _):
            t0 = 2 * u
            c0 = t0 * NW + wid

            @pl.when(c0 < NFULL)
            def _():
                step(t0, c0, in0, ob0, in1)

            @pl.when(c0 + NW < NFULL)
            def _():
                step(t0 + 1, c0 + NW, in1, ob1, in0)

            return 0

        lax.fori_loop(0, (NT + 1) // 2, loop_body, 0)

        wait_out(wid, ob0)
        wait_out(wid, ob1)

        @pl.when(wid == 4)
        def _():
            pltpu.sync_copy(t_hbm.at[:, pl.ds(TAIL_C, TAIL_N)], tail_in)
            pack_pass(tail_in, TAIL_N // 16)
            col_pass(ob0, TAIL_N)
            pltpu.sync_copy(ob0.at[pl.ds(0, TAIL_N * PH)],
                            out_hbm.at[pl.ds(TAIL_C * PH, TAIL_N * PH)])

    return body(tableT)


def _sc_emb_sum(ids2d, table):
    mesh = plsc.VectorSubcoreMesh(core_axis_name="c", subcore_axis_name="s")

    @functools.partial(
        pl.kernel,
        mesh=mesh,
        out_type=jax.ShapeDtypeStruct((B, H), jnp.float32),
        compiler_params=pltpu.CompilerParams(use_tc_tiling_on_sc=False),
        scratch_types=[
            pltpu.VMEM((2 * SPW, G), jnp.int32),
            pltpu.VMEM((G, PH), jnp.int32),
            pltpu.VMEM((G, PH), jnp.int32),
            pltpu.VMEM((G, PH), jnp.int32),
            pltpu.VMEM((G, PH), jnp.int32),
            pltpu.VMEM((SPW, H), jnp.float32),
            pltpu.SemaphoreType.DMA,
            pltpu.SemaphoreType.DMA,
        ],
    )
    def body(ids_hbm, table_hbm, out_hbm, ids_v, r0a, r0b, r1a, r1b, sums_v,
             sem0, sem1):
        wid = lax.axis_index("s") * NC + lax.axis_index("c")
        base = wid * (2 * SPW)
        pltpu.sync_copy(ids_hbm.at[pl.ds(base, 2 * SPW)], ids_v)

        def fire(s, ra, rb, sem):
            pltpu.async_copy(table_hbm.at[ids_v.at[2 * s]], ra, sem)
            pltpu.async_copy(table_hbm.at[ids_v.at[2 * s + 1]], rb, sem)

        def drain(s, ra, rb, sem):
            pltpu.make_async_copy(table_hbm.at[ids_v.at[2 * s]], ra, sem).wait()
            pltpu.make_async_copy(table_hbm.at[ids_v.at[2 * s + 1]], rb,
                                  sem).wait()

        himask = jnp.int32(-65536)

        def accum(buf, carry):
            def unpack2(w):
                lo = lax.bitcast_convert_type(lax.shift_left(w, 16),
                                              jnp.float32)
                hi = lax.bitcast_convert_type(w & himask, jnp.float32)
                return lo, hi

            def inner(i, c):
                a0, a1, a2, a3 = c
                for j in range(0, 20, 2):
                    r = i * 20 + j
                    lo0, hi0 = unpack2(buf[r, pl.ds(0, PH)])
                    lo1, hi1 = unpack2(buf[r + 1, pl.ds(0, PH)])
                    a0 = a0 + lo0
                    a1 = a1 + hi0
                    a2 = a2 + lo1
                    a3 = a3 + hi1
                return (a0, a1, a2, a3)

            return lax.fori_loop(0, G // 20, inner, carry)

        def do_sample(s, ra, rb, sem):
            drain(s, ra, rb, sem)
            z = jnp.zeros((HALF,), jnp.float32)
            c = accum(ra, (z, z, z, z))
            c = accum(rb, c)
            sums_v[s, pl.ds(0, HALF)] = c[0] + c[2]
            sums_v[s, pl.ds(HALF, HALF)] = c[1] + c[3]

        fire(0, r0a, r0b, sem0)

        def loop_body(t, _):
            s0 = 2 * t
            fire(s0 + 1, r1a, r1b, sem1)
            do_sample(s0, r0a, r0b, sem0)

            @pl.when(s0 + 2 < SPW)
            def _():
                fire(s0 + 2, r0a, r0b, sem0)

            do_sample(s0 + 1, r1a, r1b, sem1)
            return 0

        lax.fori_loop(0, SPW // 2, loop_body, 0)
        pltpu.sync_copy(sums_v, out_hbm.at[pl.ds(wid * SPW, SPW)])

    return body(ids2d, table)


def _tc_head(sums, tt, type_table, Wp, bp):
    def body(sums_ref, tt_ref, type_ref, w_ref, b_ref, out_ref):
        c1 = jnp.sum(tt_ref[...].astype(jnp.float32), axis=1, keepdims=True)
        t0 = type_ref[0:1, :]
        t1 = type_ref[1:2, :]
        inv_s = jnp.float32(1.0 / S)
        pooled = (sums_ref[...] + (jnp.float32(S) - c1) * t0 + c1 * t1) * inv_s
        out_ref[...] = (
            jnp.dot(pooled, w_ref[...], preferred_element_type=jnp.float32)
            + b_ref[...]
        )

    return pl.pallas_call(
        body,
        out_shape=jax.ShapeDtypeStruct((B, 128), jnp.float32),
    )(sums, tt, type_table, Wp, bp)


def kernel(input_ids, attention_mask, token_type_ids, emb_table, type_table,
           W, b):
    del attention_mask
    t4 = _sc_detile(emb_table.T)
    table_lin = t4.reshape(V, PH)
    ids2d = input_ids.astype(jnp.int32).reshape(NW * 2 * SPW, G)
    sums = _sc_emb_sum(ids2d, table_lin)
    Wp = jnp.pad(W.astype(jnp.float32), ((0, 0), (0, 128 - L)))
    bp = jnp.pad(b.astype(jnp.float32), (0, 128 - L)).reshape(1, 128)
    logits = _tc_head(sums, token_type_ids.astype(jnp.int32), type_table, Wp,
                      bp)
    return logits[:, :L]
